# baseline (speedup 1.0000x reference)
import functools
import math

import jax
import jax.numpy as jnp
from jax import lax
from jax.experimental import pallas as pl
from jax.experimental.pallas import tpu as pltpu

SEQ = 2048
BATCH = 2
D_MODEL = 768
NHEAD = 12
DHEAD = D_MODEL // NHEAD
NUM_LAYERS = 2
D_FF = 2048
NUM_EXPERTS = 64
LN_EPS = 1e-5


def _dot_t(a, w):
    return lax.dot_general(a, w, (((1,), (1,)), ((), ())),
                           preferred_element_type=jnp.float32)


def _mm_bias_kernel(a_ref, w_ref, b_ref, o_ref, *, relu):
    acc = _dot_t(a_ref[...], w_ref[...]) + b_ref[...]
    if relu:
        acc = jnp.maximum(acc, 0.0)
    o_ref[...] = acc


def _mm_bias(a, w, b, relu=False, bm=512):
    M, K = a.shape
    N = w.shape[0]
    return pl.pallas_call(
        functools.partial(_mm_bias_kernel, relu=relu),
        grid=(M // bm,),
        in_specs=[
            pl.BlockSpec((bm, K), lambda i: (i, 0)),
            pl.BlockSpec((N, K), lambda i: (0, 0)),
            pl.BlockSpec((1, N), lambda i: (0, 0)),
        ],
        out_specs=pl.BlockSpec((bm, N), lambda i: (i, 0)),
        out_shape=jax.ShapeDtypeStruct((M, N), jnp.float32),
    )(a, w, b.reshape(1, N))


def _attn_kernel(q_ref, k_ref, v_ref, o_ref):
    q = q_ref[0]
    k = k_ref[0]
    v = v_ref[0]
    s = _dot_t(q, k) * (1.0 / math.sqrt(DHEAD))
    m = jnp.max(s, axis=-1, keepdims=True)
    p = jnp.exp(s - m)
    p = p / jnp.sum(p, axis=-1, keepdims=True)
    o_ref[0] = jnp.dot(p, v, preferred_element_type=jnp.float32)


def _attn(q, k, v, bq=256):
    BH, S, DH = q.shape
    return pl.pallas_call(
        _attn_kernel,
        grid=(BH, S // bq),
        in_specs=[
            pl.BlockSpec((1, bq, DH), lambda i, j: (i, j, 0)),
            pl.BlockSpec((1, S, DH), lambda i, j: (i, 0, 0)),
            pl.BlockSpec((1, S, DH), lambda i, j: (i, 0, 0)),
        ],
        out_specs=pl.BlockSpec((1, bq, DH), lambda i, j: (i, j, 0)),
        out_shape=jax.ShapeDtypeStruct((BH, S, DH), jnp.float32),
    )(q, k, v)


def _mm_res_ln_kernel(a_ref, w_ref, b_ref, r_ref, g_ref, bb_ref, o_ref):
    y = _dot_t(a_ref[...], w_ref[...]) + b_ref[...] + r_ref[...]
    m = jnp.mean(y, axis=-1, keepdims=True)
    c = y - m
    v = jnp.mean(c * c, axis=-1, keepdims=True)
    o_ref[...] = c * lax.rsqrt(v + LN_EPS) * g_ref[...] + bb_ref[...]


def _mm_res_ln(a, w, b, res, g, beta, bm=512):
    M, K = a.shape
    N = w.shape[0]
    return pl.pallas_call(
        _mm_res_ln_kernel,
        grid=(M // bm,),
        in_specs=[
            pl.BlockSpec((bm, K), lambda i: (i, 0)),
            pl.BlockSpec((N, K), lambda i: (0, 0)),
            pl.BlockSpec((1, N), lambda i: (0, 0)),
            pl.BlockSpec((bm, N), lambda i: (i, 0)),
            pl.BlockSpec((1, N), lambda i: (0, 0)),
            pl.BlockSpec((1, N), lambda i: (0, 0)),
        ],
        out_specs=pl.BlockSpec((bm, N), lambda i: (i, 0)),
        out_shape=jax.ShapeDtypeStruct((M, N), jnp.float32),
    )(a, w, b.reshape(1, N), res, g.reshape(1, N), beta.reshape(1, N))


def _colmean_kernel(h_ref, o_ref):
    o_ref[...] = jnp.mean(h_ref[...], axis=0, keepdims=True)


def _colmean(h2):
    S, BD = h2.shape
    return pl.pallas_call(
        _colmean_kernel,
        grid=(1,),
        in_specs=[pl.BlockSpec((S, BD), lambda i: (0, 0))],
        out_specs=pl.BlockSpec((1, BD), lambda i: (0, 0)),
        out_shape=jax.ShapeDtypeStruct((1, BD), jnp.float32),
    )(h2)


def _gate_kernel(rep_ref, wg_ref, bg_ref, gw_ref, idx_ref):
    lg = _dot_t(rep_ref[...], wg_ref[...]) + bg_ref[...]
    m = jnp.max(lg, axis=-1, keepdims=True)
    e = jnp.exp(lg - m)
    gw_ref[...] = e / jnp.sum(e, axis=-1, keepdims=True)
    idx_ref[...] = jnp.argmax(lg, axis=-1, keepdims=True).astype(jnp.int32)


def _gate(rep, Wg, bg):
    B, d = rep.shape
    E = Wg.shape[0]
    return pl.pallas_call(
        _gate_kernel,
        grid=(1,),
        in_specs=[
            pl.BlockSpec((B, d), lambda i: (0, 0)),
            pl.BlockSpec((E, d), lambda i: (0, 0)),
            pl.BlockSpec((1, E), lambda i: (0, 0)),
        ],
        out_specs=[
            pl.BlockSpec((B, E), lambda i: (0, 0)),
            pl.BlockSpec((B, 1), lambda i: (0, 0)),
        ],
        out_shape=[
            jax.ShapeDtypeStruct((B, E), jnp.float32),
            jax.ShapeDtypeStruct((B, 1), jnp.int32),
        ],
    )(rep, Wg, bg.reshape(1, E))


def _expert_kernel(idx_ref, rep_ref, we_ref, be_ref, o_ref):
    o_ref[0] = _dot_t(rep_ref[0], we_ref[0]) + be_ref[0]


def _expert(idx, rep, We, be):
    B, d = rep.shape
    E = We.shape[0]
    grid_spec = pltpu.PrefetchScalarGridSpec(
        num_scalar_prefetch=1,
        grid=(B,),
        in_specs=[
            pl.BlockSpec((1, 1, d), lambda i, idx_ref: (i, 0, 0)),
            pl.BlockSpec((1, d, d), lambda i, idx_ref: (idx_ref[i], 0, 0)),
            pl.BlockSpec((1, 1, d), lambda i, idx_ref: (idx_ref[i], 0, 0)),
        ],
        out_specs=pl.BlockSpec((1, 1, d), lambda i, idx_ref: (i, 0, 0)),
    )
    out = pl.pallas_call(
        _expert_kernel,
        grid_spec=grid_spec,
        out_shape=jax.ShapeDtypeStruct((B, 1, d), jnp.float32),
    )(idx, rep.reshape(B, 1, d), We, be.reshape(E, 1, d))
    return out.reshape(B, d)


def _head(eo, Wh, bh, bn=4096):
    B, K = eo.shape
    V = Wh.shape[0]
    return pl.pallas_call(
        functools.partial(_mm_bias_kernel, relu=False),
        grid=(pl.cdiv(V, bn),),
        in_specs=[
            pl.BlockSpec((B, K), lambda j: (0, 0)),
            pl.BlockSpec((bn, K), lambda j: (j, 0)),
            pl.BlockSpec((1, bn), lambda j: (0, j)),
        ],
        out_specs=pl.BlockSpec((B, bn), lambda j: (0, j)),
        out_shape=jax.ShapeDtypeStruct((B, V), jnp.float32),
    )(eo, Wh, bh.reshape(1, V))


def kernel(x, Wqkv, bqkv, Wo, bo, ln1g, ln1b, W1, b1, W2, b2, ln2g, ln2b,
           Wg, bg, We, be, Wh, bh):
    S, B, d = x.shape
    h = x.reshape(S * B, d)
    for i in range(NUM_LAYERS):
        qkv = _mm_bias(h, Wqkv[i], bqkv[i])
        qkvh = (qkv.reshape(S, B, 3, NHEAD, DHEAD)
                   .transpose(2, 1, 3, 0, 4)
                   .reshape(3, B * NHEAD, S, DHEAD))
        ao = _attn(qkvh[0], qkvh[1], qkvh[2])
        ao = (ao.reshape(B, NHEAD, S, DHEAD)
                .transpose(2, 0, 1, 3)
                .reshape(S * B, d))
        h = _mm_res_ln(ao, Wo[i], bo[i], h, ln1g[i], ln1b[i])
        f = _mm_bias(h, W1[i], b1[i], relu=True)
        h = _mm_res_ln(f, W2[i], b2[i], h, ln2g[i], ln2b[i])
    rep = _colmean(h.reshape(S, B * d)).reshape(B, d)
    gw, idx2 = _gate(rep, Wg, bg)
    idx = idx2.reshape(B)
    eo = _expert(idx, rep, We, be)
    logits = _head(eo, Wh, bh)
    return logits, gw, idx


# speedup vs baseline: 2.2302x; 2.2302x over previous
import functools
import math

import jax
import jax.numpy as jnp
from jax import lax
from jax.experimental import pallas as pl
from jax.experimental.pallas import tpu as pltpu

SEQ = 2048
BATCH = 2
D_MODEL = 768
NHEAD = 12
DHEAD = D_MODEL // NHEAD
NUM_LAYERS = 2
D_FF = 2048
NUM_EXPERTS = 64
LN_EPS = 1e-5


def _dot_t(a, w):
    return lax.dot_general(a, w, (((1,), (1,)), ((), ())),
                           preferred_element_type=jnp.float32)


def _mm_bias_kernel(a_ref, w_ref, b_ref, o_ref, *, relu):
    acc = _dot_t(a_ref[...], w_ref[...]) + b_ref[...]
    if relu:
        acc = jnp.maximum(acc, 0.0)
    o_ref[...] = acc


def _mm_bias(a, w, b, relu=False, bm=512):
    M, K = a.shape
    N = w.shape[0]
    return pl.pallas_call(
        functools.partial(_mm_bias_kernel, relu=relu),
        grid=(M // bm,),
        in_specs=[
            pl.BlockSpec((bm, K), lambda i: (i, 0)),
            pl.BlockSpec((N, K), lambda i: (0, 0)),
            pl.BlockSpec((1, N), lambda i: (0, 0)),
        ],
        out_specs=pl.BlockSpec((bm, N), lambda i: (i, 0)),
        out_shape=jax.ShapeDtypeStruct((M, N), jnp.float32),
    )(a, w, b.reshape(1, N))


def _mm_bias_bmajor(x2, w, b, bm=512):
    S, BD = x2.shape
    K = BD // BATCH
    N = w.shape[0]
    sblocks = S // bm
    return pl.pallas_call(
        functools.partial(_mm_bias_kernel, relu=False),
        grid=(BATCH, sblocks),
        in_specs=[
            pl.BlockSpec((bm, K), lambda bb, j: (j, bb)),
            pl.BlockSpec((N, K), lambda bb, j: (0, 0)),
            pl.BlockSpec((1, N), lambda bb, j: (0, 0)),
        ],
        out_specs=pl.BlockSpec((bm, N), lambda bb, j: (bb * sblocks + j, 0)),
        out_shape=jax.ShapeDtypeStruct((BATCH * S, N), jnp.float32),
    )(x2, w, b.reshape(1, N))


def _attn_kernel(q_ref, k_ref, v_ref, o_ref):
    qq = q_ref[...] * (1.0 / math.sqrt(DHEAD))
    kk = k_ref[...]
    vv = v_ref[...]
    outs = []
    for t in (0, 1):
        q = qq[:, t * DHEAD:(t + 1) * DHEAD]
        k = kk[:, t * DHEAD:(t + 1) * DHEAD]
        v = vv[:, t * DHEAD:(t + 1) * DHEAD]
        e = jnp.exp(_dot_t(q, k))
        den = jnp.sum(e, axis=-1, keepdims=True)
        o = jnp.dot(e, v, preferred_element_type=jnp.float32)
        outs.append(o / den)
    o_ref[...] = jnp.concatenate(outs, axis=-1)


def _attn(qkv, bq=256):
    BS = qkv.shape[0]
    S = BS // BATCH
    sblocks = S // bq
    npair = NHEAD // 2
    return pl.pallas_call(
        _attn_kernel,
        grid=(BATCH, npair, sblocks),
        in_specs=[
            pl.BlockSpec((bq, 2 * DHEAD),
                         lambda bb, p, j: (bb * sblocks + j, p)),
            pl.BlockSpec((S, 2 * DHEAD), lambda bb, p, j: (bb, npair + p)),
            pl.BlockSpec((S, 2 * DHEAD), lambda bb, p, j: (bb, 2 * npair + p)),
        ],
        out_specs=pl.BlockSpec((bq, 2 * DHEAD),
                               lambda bb, p, j: (bb * sblocks + j, p)),
        out_shape=jax.ShapeDtypeStruct((BS, D_MODEL), jnp.float32),
    )(qkv, qkv, qkv)


def _mm_res_ln_kernel(a_ref, w_ref, b_ref, r_ref, g_ref, bb_ref, o_ref):
    y = _dot_t(a_ref[...], w_ref[...]) + b_ref[...] + r_ref[...]
    m = jnp.mean(y, axis=-1, keepdims=True)
    c = y - m
    v = jnp.mean(c * c, axis=-1, keepdims=True)
    o_ref[...] = c * lax.rsqrt(v + LN_EPS) * g_ref[...] + bb_ref[...]


def _mm_res_ln(a, w, b, res, g, beta, res_is_sbview=False, bm=512):
    M, K = a.shape
    N = w.shape[0]
    sblocks = (M // BATCH) // bm
    if res_is_sbview:
        res_spec = pl.BlockSpec((bm, N), lambda bb, j: (j, bb))
    else:
        res_spec = pl.BlockSpec((bm, N), lambda bb, j: (bb * sblocks + j, 0))
    return pl.pallas_call(
        _mm_res_ln_kernel,
        grid=(BATCH, sblocks),
        in_specs=[
            pl.BlockSpec((bm, K), lambda bb, j: (bb * sblocks + j, 0)),
            pl.BlockSpec((N, K), lambda bb, j: (0, 0)),
            pl.BlockSpec((1, N), lambda bb, j: (0, 0)),
            res_spec,
            pl.BlockSpec((1, N), lambda bb, j: (0, 0)),
            pl.BlockSpec((1, N), lambda bb, j: (0, 0)),
        ],
        out_specs=pl.BlockSpec((bm, N), lambda bb, j: (bb * sblocks + j, 0)),
        out_shape=jax.ShapeDtypeStruct((M, N), jnp.float32),
    )(a, w, b.reshape(1, N), res, g.reshape(1, N), beta.reshape(1, N))


def _colmean_kernel(h_ref, o_ref):
    o_ref[0] = jnp.mean(h_ref[...], axis=0, keepdims=True)


def _colmean(h):
    BS, d = h.shape
    S = BS // BATCH
    out = pl.pallas_call(
        _colmean_kernel,
        grid=(BATCH,),
        in_specs=[pl.BlockSpec((S, d), lambda bb: (bb, 0))],
        out_specs=pl.BlockSpec((1, 1, d), lambda bb: (bb, 0, 0)),
        out_shape=jax.ShapeDtypeStruct((BATCH, 1, d), jnp.float32),
    )(h)
    return out.reshape(BATCH, d)


def _gate_kernel(rep_ref, wg_ref, bg_ref, gw_ref, idx_ref):
    lg = _dot_t(rep_ref[...], wg_ref[...]) + bg_ref[...]
    m = jnp.max(lg, axis=-1, keepdims=True)
    e = jnp.exp(lg - m)
    gw_ref[...] = e / jnp.sum(e, axis=-1, keepdims=True)
    idx_ref[...] = jnp.argmax(lg, axis=-1, keepdims=True).astype(jnp.int32)


def _gate(rep, Wg, bg):
    B, d = rep.shape
    E = Wg.shape[0]
    return pl.pallas_call(
        _gate_kernel,
        grid=(1,),
        in_specs=[
            pl.BlockSpec((B, d), lambda i: (0, 0)),
            pl.BlockSpec((E, d), lambda i: (0, 0)),
            pl.BlockSpec((1, E), lambda i: (0, 0)),
        ],
        out_specs=[
            pl.BlockSpec((B, E), lambda i: (0, 0)),
            pl.BlockSpec((B, 1), lambda i: (0, 0)),
        ],
        out_shape=[
            jax.ShapeDtypeStruct((B, E), jnp.float32),
            jax.ShapeDtypeStruct((B, 1), jnp.int32),
        ],
    )(rep, Wg, bg.reshape(1, E))


def _expert_kernel(idx_ref, rep_ref, we_ref, be_ref, o_ref):
    o_ref[0] = _dot_t(rep_ref[0], we_ref[0]) + be_ref[0]


def _expert(idx, rep, We, be):
    B, d = rep.shape
    E = We.shape[0]
    grid_spec = pltpu.PrefetchScalarGridSpec(
        num_scalar_prefetch=1,
        grid=(B,),
        in_specs=[
            pl.BlockSpec((1, 1, d), lambda i, idx_ref: (i, 0, 0)),
            pl.BlockSpec((1, d, d), lambda i, idx_ref: (idx_ref[i], 0, 0)),
            pl.BlockSpec((1, 1, d), lambda i, idx_ref: (idx_ref[i], 0, 0)),
        ],
        out_specs=pl.BlockSpec((1, 1, d), lambda i, idx_ref: (i, 0, 0)),
    )
    out = pl.pallas_call(
        _expert_kernel,
        grid_spec=grid_spec,
        out_shape=jax.ShapeDtypeStruct((B, 1, d), jnp.float32),
    )(idx, rep.reshape(B, 1, d), We, be.reshape(E, 1, d))
    return out.reshape(B, d)


def _head(eo, Wh, bh, bn=4096):
    B, K = eo.shape
    V = Wh.shape[0]
    return pl.pallas_call(
        functools.partial(_mm_bias_kernel, relu=False),
        grid=(pl.cdiv(V, bn),),
        in_specs=[
            pl.BlockSpec((B, K), lambda j: (0, 0)),
            pl.BlockSpec((bn, K), lambda j: (j, 0)),
            pl.BlockSpec((1, bn), lambda j: (0, j)),
        ],
        out_specs=pl.BlockSpec((B, bn), lambda j: (0, j)),
        out_shape=jax.ShapeDtypeStruct((B, V), jnp.float32),
    )(eo, Wh, bh.reshape(1, V))


def kernel(x, Wqkv, bqkv, Wo, bo, ln1g, ln1b, W1, b1, W2, b2, ln2g, ln2b,
           Wg, bg, We, be, Wh, bh):
    S, B, d = x.shape
    x2 = x.reshape(S, B * d)
    h = None
    for i in range(NUM_LAYERS):
        if i == 0:
            qkv = _mm_bias_bmajor(x2, Wqkv[i], bqkv[i])
        else:
            qkv = _mm_bias(h, Wqkv[i], bqkv[i])
        ao = _attn(qkv)
        if i == 0:
            h = _mm_res_ln(ao, Wo[i], bo[i], x2, ln1g[i], ln1b[i],
                           res_is_sbview=True)
        else:
            h = _mm_res_ln(ao, Wo[i], bo[i], h, ln1g[i], ln1b[i])
        f = _mm_bias(h, W1[i], b1[i], relu=True)
        h = _mm_res_ln(f, W2[i], b2[i], h, ln2g[i], ln2b[i])
    rep = _colmean(h)
    gw, idx2 = _gate(rep, Wg, bg)
    idx = idx2.reshape(B)
    eo = _expert(idx, rep, We, be)
    logits = _head(eo, Wh, bh)
    return logits, gw, idx


# speedup vs baseline: 2.2406x; 1.0047x over previous
import functools
import math

import jax
import jax.numpy as jnp
from jax import lax
from jax.experimental import pallas as pl
from jax.experimental.pallas import tpu as pltpu

SEQ = 2048
BATCH = 2
D_MODEL = 768
NHEAD = 12
DHEAD = D_MODEL // NHEAD
NUM_LAYERS = 2
D_FF = 2048
NUM_EXPERTS = 64
LN_EPS = 1e-5


def _dot_t(a, w):
    return lax.dot_general(a, w, (((1,), (1,)), ((), ())),
                           preferred_element_type=jnp.float32)


def _mm_bias_kernel(a_ref, w_ref, b_ref, o_ref, *, relu):
    acc = _dot_t(a_ref[...], w_ref[...]) + b_ref[...]
    if relu:
        acc = jnp.maximum(acc, 0.0)
    o_ref[...] = acc


def _mm_bias(a, w, b, relu=False, bm=512):
    M, K = a.shape
    N = w.shape[0]
    return pl.pallas_call(
        functools.partial(_mm_bias_kernel, relu=relu),
        grid=(M // bm,),
        in_specs=[
            pl.BlockSpec((bm, K), lambda i: (i, 0)),
            pl.BlockSpec((N, K), lambda i: (0, 0)),
            pl.BlockSpec((1, N), lambda i: (0, 0)),
        ],
        out_specs=pl.BlockSpec((bm, N), lambda i: (i, 0)),
        out_shape=jax.ShapeDtypeStruct((M, N), jnp.float32),
    )(a, w, b.reshape(1, N))


def _mm_bias_bmajor(x2, w, b, bm=512):
    S, BD = x2.shape
    K = BD // BATCH
    N = w.shape[0]
    sblocks = S // bm
    return pl.pallas_call(
        functools.partial(_mm_bias_kernel, relu=False),
        grid=(BATCH, sblocks),
        in_specs=[
            pl.BlockSpec((bm, K), lambda bb, j: (j, bb)),
            pl.BlockSpec((N, K), lambda bb, j: (0, 0)),
            pl.BlockSpec((1, N), lambda bb, j: (0, 0)),
        ],
        out_specs=pl.BlockSpec((bm, N), lambda bb, j: (bb * sblocks + j, 0)),
        out_shape=jax.ShapeDtypeStruct((BATCH * S, N), jnp.float32),
    )(x2, w, b.reshape(1, N))


def _attn_kernel(q_ref, k_ref, v_ref, o_ref):
    qq = q_ref[...] * (1.0 / math.sqrt(DHEAD))
    kk = k_ref[...]
    vv = v_ref[...]
    outs = []
    for t in (0, 1):
        q = qq[:, t * DHEAD:(t + 1) * DHEAD].astype(jnp.bfloat16)
        k = kk[:, t * DHEAD:(t + 1) * DHEAD].astype(jnp.bfloat16)
        v = vv[:, t * DHEAD:(t + 1) * DHEAD].astype(jnp.bfloat16)
        e = jnp.exp(_dot_t(q, k))
        den = jnp.sum(e, axis=-1, keepdims=True)
        o = jnp.dot(e.astype(jnp.bfloat16), v,
                    preferred_element_type=jnp.float32)
        outs.append(o / den)
    o_ref[...] = jnp.concatenate(outs, axis=-1)


def _attn(qkv, bq=256):
    BS = qkv.shape[0]
    S = BS // BATCH
    sblocks = S // bq
    npair = NHEAD // 2
    return pl.pallas_call(
        _attn_kernel,
        grid=(BATCH, npair, sblocks),
        in_specs=[
            pl.BlockSpec((bq, 2 * DHEAD),
                         lambda bb, p, j: (bb * sblocks + j, p)),
            pl.BlockSpec((S, 2 * DHEAD), lambda bb, p, j: (bb, npair + p)),
            pl.BlockSpec((S, 2 * DHEAD), lambda bb, p, j: (bb, 2 * npair + p)),
        ],
        out_specs=pl.BlockSpec((bq, 2 * DHEAD),
                               lambda bb, p, j: (bb * sblocks + j, p)),
        out_shape=jax.ShapeDtypeStruct((BS, D_MODEL), jnp.float32),
    )(qkv, qkv, qkv)


def _mm_res_ln_kernel(a_ref, w_ref, b_ref, r_ref, g_ref, bb_ref, o_ref):
    y = _dot_t(a_ref[...], w_ref[...]) + b_ref[...] + r_ref[...]
    m = jnp.mean(y, axis=-1, keepdims=True)
    c = y - m
    v = jnp.mean(c * c, axis=-1, keepdims=True)
    o_ref[...] = c * lax.rsqrt(v + LN_EPS) * g_ref[...] + bb_ref[...]


def _mm_res_ln(a, w, b, res, g, beta, res_is_sbview=False, bm=512):
    M, K = a.shape
    N = w.shape[0]
    sblocks = (M // BATCH) // bm
    if res_is_sbview:
        res_spec = pl.BlockSpec((bm, N), lambda bb, j: (j, bb))
    else:
        res_spec = pl.BlockSpec((bm, N), lambda bb, j: (bb * sblocks + j, 0))
    return pl.pallas_call(
        _mm_res_ln_kernel,
        grid=(BATCH, sblocks),
        in_specs=[
            pl.BlockSpec((bm, K), lambda bb, j: (bb * sblocks + j, 0)),
            pl.BlockSpec((N, K), lambda bb, j: (0, 0)),
            pl.BlockSpec((1, N), lambda bb, j: (0, 0)),
            res_spec,
            pl.BlockSpec((1, N), lambda bb, j: (0, 0)),
            pl.BlockSpec((1, N), lambda bb, j: (0, 0)),
        ],
        out_specs=pl.BlockSpec((bm, N), lambda bb, j: (bb * sblocks + j, 0)),
        out_shape=jax.ShapeDtypeStruct((M, N), jnp.float32),
    )(a, w, b.reshape(1, N), res, g.reshape(1, N), beta.reshape(1, N))


def _colmean_kernel(h_ref, o_ref):
    o_ref[0] = jnp.mean(h_ref[...], axis=0, keepdims=True)


def _colmean(h):
    BS, d = h.shape
    S = BS // BATCH
    out = pl.pallas_call(
        _colmean_kernel,
        grid=(BATCH,),
        in_specs=[pl.BlockSpec((S, d), lambda bb: (bb, 0))],
        out_specs=pl.BlockSpec((1, 1, d), lambda bb: (bb, 0, 0)),
        out_shape=jax.ShapeDtypeStruct((BATCH, 1, d), jnp.float32),
    )(h)
    return out.reshape(BATCH, d)


def _gate_kernel(rep_ref, wg_ref, bg_ref, gw_ref, idx_ref):
    lg = _dot_t(rep_ref[...], wg_ref[...]) + bg_ref[...]
    m = jnp.max(lg, axis=-1, keepdims=True)
    e = jnp.exp(lg - m)
    gw_ref[...] = e / jnp.sum(e, axis=-1, keepdims=True)
    idx_ref[...] = jnp.argmax(lg, axis=-1, keepdims=True).astype(jnp.int32)


def _gate(rep, Wg, bg):
    B, d = rep.shape
    E = Wg.shape[0]
    return pl.pallas_call(
        _gate_kernel,
        grid=(1,),
        in_specs=[
            pl.BlockSpec((B, d), lambda i: (0, 0)),
            pl.BlockSpec((E, d), lambda i: (0, 0)),
            pl.BlockSpec((1, E), lambda i: (0, 0)),
        ],
        out_specs=[
            pl.BlockSpec((B, E), lambda i: (0, 0)),
            pl.BlockSpec((B, 1), lambda i: (0, 0)),
        ],
        out_shape=[
            jax.ShapeDtypeStruct((B, E), jnp.float32),
            jax.ShapeDtypeStruct((B, 1), jnp.int32),
        ],
    )(rep, Wg, bg.reshape(1, E))


def _expert_kernel(idx_ref, rep_ref, we_ref, be_ref, o_ref):
    o_ref[0] = _dot_t(rep_ref[0], we_ref[0]) + be_ref[0]


def _expert(idx, rep, We, be):
    B, d = rep.shape
    E = We.shape[0]
    grid_spec = pltpu.PrefetchScalarGridSpec(
        num_scalar_prefetch=1,
        grid=(B,),
        in_specs=[
            pl.BlockSpec((1, 1, d), lambda i, idx_ref: (i, 0, 0)),
            pl.BlockSpec((1, d, d), lambda i, idx_ref: (idx_ref[i], 0, 0)),
            pl.BlockSpec((1, 1, d), lambda i, idx_ref: (idx_ref[i], 0, 0)),
        ],
        out_specs=pl.BlockSpec((1, 1, d), lambda i, idx_ref: (i, 0, 0)),
    )
    out = pl.pallas_call(
        _expert_kernel,
        grid_spec=grid_spec,
        out_shape=jax.ShapeDtypeStruct((B, 1, d), jnp.float32),
    )(idx, rep.reshape(B, 1, d), We, be.reshape(E, 1, d))
    return out.reshape(B, d)


def _head(eo, Wh, bh, bn=4096):
    B, K = eo.shape
    V = Wh.shape[0]
    return pl.pallas_call(
        functools.partial(_mm_bias_kernel, relu=False),
        grid=(pl.cdiv(V, bn),),
        in_specs=[
            pl.BlockSpec((B, K), lambda j: (0, 0)),
            pl.BlockSpec((bn, K), lambda j: (j, 0)),
            pl.BlockSpec((1, bn), lambda j: (0, j)),
        ],
        out_specs=pl.BlockSpec((B, bn), lambda j: (0, j)),
        out_shape=jax.ShapeDtypeStruct((B, V), jnp.float32),
    )(eo, Wh, bh.reshape(1, V))


def kernel(x, Wqkv, bqkv, Wo, bo, ln1g, ln1b, W1, b1, W2, b2, ln2g, ln2b,
           Wg, bg, We, be, Wh, bh):
    S, B, d = x.shape
    x2 = x.reshape(S, B * d)
    h = None
    for i in range(NUM_LAYERS):
        if i == 0:
            qkv = _mm_bias_bmajor(x2, Wqkv[i], bqkv[i])
        else:
            qkv = _mm_bias(h, Wqkv[i], bqkv[i])
        ao = _attn(qkv)
        if i == 0:
            h = _mm_res_ln(ao, Wo[i], bo[i], x2, ln1g[i], ln1b[i],
                           res_is_sbview=True)
        else:
            h = _mm_res_ln(ao, Wo[i], bo[i], h, ln1g[i], ln1b[i])
        f = _mm_bias(h, W1[i], b1[i], relu=True)
        h = _mm_res_ln(f, W2[i], b2[i], h, ln2g[i], ln2b[i])
    rep = _colmean(h)
    gw, idx2 = _gate(rep, Wg, bg)
    idx = idx2.reshape(B)
    eo = _expert(idx, rep, We, be)
    logits = _head(eo, Wh, bh)
    return logits, gw, idx


# speedup vs baseline: 2.5847x; 1.1536x over previous
import functools
import math

import jax
import jax.numpy as jnp
from jax import lax
from jax.experimental import pallas as pl
from jax.experimental.pallas import tpu as pltpu

SEQ = 2048
BATCH = 2
D_MODEL = 768
NHEAD = 12
DHEAD = D_MODEL // NHEAD
NUM_LAYERS = 2
D_FF = 2048
NUM_EXPERTS = 64
LN_EPS = 1e-5


def _dot_t(a, w):
    return lax.dot_general(a, w, (((1,), (1,)), ((), ())),
                           preferred_element_type=jnp.float32)


def _mm_bias_kernel(a_ref, w_ref, b_ref, o_ref, *, relu):
    acc = _dot_t(a_ref[...], w_ref[...]) + b_ref[...]
    if relu:
        acc = jnp.maximum(acc, 0.0)
    o_ref[...] = acc


def _mm_bias(a, w, b, relu=False, bm=512):
    M, K = a.shape
    N = w.shape[0]
    return pl.pallas_call(
        functools.partial(_mm_bias_kernel, relu=relu),
        grid=(M // bm,),
        in_specs=[
            pl.BlockSpec((bm, K), lambda i: (i, 0)),
            pl.BlockSpec((N, K), lambda i: (0, 0)),
            pl.BlockSpec((1, N), lambda i: (0, 0)),
        ],
        out_specs=pl.BlockSpec((bm, N), lambda i: (i, 0)),
        out_shape=jax.ShapeDtypeStruct((M, N), jnp.float32),
    )(a, w, b.reshape(1, N))


def _mm_bias_bmajor_kernel(a_ref, w_ref, b_ref, o_ref):
    aa = a_ref[...]
    a = jnp.where(pl.program_id(0) == 0, aa[:, 0, :], aa[:, 1, :])
    o_ref[...] = _dot_t(a, w_ref[...]) + b_ref[...]


def _mm_bias_bmajor(x, w, b, bm=512):
    S, B, K = x.shape
    N = w.shape[0]
    sblocks = S // bm
    return pl.pallas_call(
        _mm_bias_bmajor_kernel,
        grid=(BATCH, sblocks),
        in_specs=[
            pl.BlockSpec((bm, B, K), lambda bb, j: (j, 0, 0)),
            pl.BlockSpec((N, K), lambda bb, j: (0, 0)),
            pl.BlockSpec((1, N), lambda bb, j: (0, 0)),
        ],
        out_specs=pl.BlockSpec((bm, N), lambda bb, j: (bb * sblocks + j, 0)),
        out_shape=jax.ShapeDtypeStruct((BATCH * S, N), jnp.float32),
    )(x, w, b.reshape(1, N))


def _attn_kernel(q_ref, k_ref, v_ref, o_ref):
    qq = q_ref[...] * (math.log2(math.e) / math.sqrt(DHEAD))
    kk = k_ref[...]
    vv = v_ref[...]
    outs = []
    for t in (0, 1):
        q = qq[:, t * DHEAD:(t + 1) * DHEAD]
        k = kk[:, t * DHEAD:(t + 1) * DHEAD]
        v = vv[:, t * DHEAD:(t + 1) * DHEAD]
        e = jnp.exp2(_dot_t(q, k))
        den = jnp.sum(e, axis=-1, keepdims=True)
        o = jnp.dot(e, v, preferred_element_type=jnp.float32)
        outs.append(o / den)
    o_ref[...] = jnp.concatenate(outs, axis=-1)


def _attn(qkv, bq=512):
    BS = qkv.shape[0]
    S = BS // BATCH
    sblocks = S // bq
    npair = NHEAD // 2
    return pl.pallas_call(
        _attn_kernel,
        grid=(BATCH, npair, sblocks),
        in_specs=[
            pl.BlockSpec((bq, 2 * DHEAD),
                         lambda bb, p, j: (bb * sblocks + j, p)),
            pl.BlockSpec((S, 2 * DHEAD), lambda bb, p, j: (bb, npair + p)),
            pl.BlockSpec((S, 2 * DHEAD), lambda bb, p, j: (bb, 2 * npair + p)),
        ],
        out_specs=pl.BlockSpec((bq, 2 * DHEAD),
                               lambda bb, p, j: (bb * sblocks + j, p)),
        out_shape=jax.ShapeDtypeStruct((BS, D_MODEL), jnp.float32),
    )(qkv, qkv, qkv)


def _ln(y, g, b):
    m = jnp.mean(y, axis=-1, keepdims=True)
    c = y - m
    v = jnp.mean(c * c, axis=-1, keepdims=True)
    return c * lax.rsqrt(v + LN_EPS) * g + b


def _block_kernel(a_ref, wo_ref, bo_ref, r_ref, g1_ref, bb1_ref, w1_ref,
                  b1_ref, w2_ref, b2_ref, g2_ref, bb2_ref, *rest, pool):
    if pool:
        wq_ref, bq_ref, rep_ref = rest
    else:
        wq_ref, bq_ref, h_ref, q_ref = rest
    r = r_ref[...]
    if r.ndim == 3:
        r = jnp.where(pl.program_id(0) == 0, r[:, 0, :], r[:, 1, :])
    h1 = _ln(_dot_t(a_ref[...], wo_ref[...]) + bo_ref[...] + r,
             g1_ref[...], bb1_ref[...])
    f = jnp.maximum(_dot_t(h1, w1_ref[...]) + b1_ref[...], 0.0)
    h = _ln(_dot_t(f, w2_ref[...]) + b2_ref[...] + h1,
            g2_ref[...], bb2_ref[...])
    if pool:
        part = jnp.sum(h, axis=0, keepdims=True) * (1.0 / SEQ)

        @pl.when(pl.program_id(1) == 0)
        def _():
            rep_ref[0] = jnp.zeros_like(rep_ref[0])

        rep_ref[0] += part
    else:
        h_ref[...] = h
        q_ref[...] = _dot_t(h, wq_ref[...]) + bq_ref[...]


def _block(a, wo, bo, res, g1, bb1, w1, b1, w2, b2, g2, bb2,
           wq=None, bq=None, res_is_sbview=False, bm=512):
    M, K = a.shape
    F = w1.shape[0]
    sblocks = (M // BATCH) // bm
    if res_is_sbview:
        res_spec = pl.BlockSpec((bm, BATCH, K), lambda bb, j: (j, 0, 0))
    else:
        res_spec = pl.BlockSpec((bm, K), lambda bb, j: (bb * sblocks + j, 0))
    row = lambda bb, j: (bb * sblocks + j, 0)
    rep = lambda bb, j: (0, 0)
    has_q = wq is not None
    NQ = wq.shape[0] if has_q else 1
    if not has_q:
        wq = jnp.zeros((1, K), jnp.float32)
        bq = jnp.zeros((1,), jnp.float32)
    if has_q:
        out_specs = [pl.BlockSpec((bm, K), row),
                     pl.BlockSpec((bm, NQ), row)]
        out_shape = [jax.ShapeDtypeStruct((M, K), jnp.float32),
                     jax.ShapeDtypeStruct((M, NQ), jnp.float32)]
    else:
        out_specs = [pl.BlockSpec((1, 1, K), lambda bb, j: (bb, 0, 0))]
        out_shape = [jax.ShapeDtypeStruct((BATCH, 1, K), jnp.float32)]
    outs = pl.pallas_call(
        functools.partial(_block_kernel, pool=not has_q),
        grid=(BATCH, sblocks),
        in_specs=[
            pl.BlockSpec((bm, K), row),
            pl.BlockSpec((K, K), rep),
            pl.BlockSpec((1, K), rep),
            res_spec,
            pl.BlockSpec((1, K), rep),
            pl.BlockSpec((1, K), rep),
            pl.BlockSpec((F, K), rep),
            pl.BlockSpec((1, F), rep),
            pl.BlockSpec((K, F), rep),
            pl.BlockSpec((1, K), rep),
            pl.BlockSpec((1, K), rep),
            pl.BlockSpec((1, K), rep),
            pl.BlockSpec((NQ, K), rep),
            pl.BlockSpec((1, NQ), rep),
        ],
        out_specs=out_specs,
        out_shape=out_shape,
    )(a, wo, bo.reshape(1, K), res, g1.reshape(1, K), bb1.reshape(1, K),
      w1, b1.reshape(1, F), w2, b2.reshape(1, K), g2.reshape(1, K),
      bb2.reshape(1, K), wq, bq.reshape(1, NQ))
    return outs if has_q else (outs[0].reshape(BATCH, K), None)


def _gate_kernel(rep_ref, wg_ref, bg_ref, gw_ref, idx_ref):
    lg = _dot_t(rep_ref[...], wg_ref[...]) + bg_ref[...]
    m = jnp.max(lg, axis=-1, keepdims=True)
    e = jnp.exp(lg - m)
    gw_ref[...] = e / jnp.sum(e, axis=-1, keepdims=True)
    idx_ref[...] = jnp.argmax(lg, axis=-1, keepdims=True).astype(jnp.int32)


def _gate(rep, Wg, bg):
    B, d = rep.shape
    E = Wg.shape[0]
    gw, idx = pl.pallas_call(
        _gate_kernel,
        out_shape=[
            jax.ShapeDtypeStruct((B, E), jnp.float32),
            jax.ShapeDtypeStruct((B, 1), jnp.int32),
        ],
    )(rep, Wg, bg.reshape(1, E))
    return gw, idx.reshape(B)


def _expert_kernel(idx_ref, rep_ref, we_ref, be_ref, o_ref):
    o_ref[0] = _dot_t(rep_ref[0], we_ref[0]) + be_ref[0]


def _expert(idx, rep, We, be):
    B, d = rep.shape
    E = We.shape[0]
    grid_spec = pltpu.PrefetchScalarGridSpec(
        num_scalar_prefetch=1,
        grid=(B,),
        in_specs=[
            pl.BlockSpec((1, 1, d), lambda i, idx_ref: (i, 0, 0)),
            pl.BlockSpec((1, d, d), lambda i, idx_ref: (idx_ref[i], 0, 0)),
            pl.BlockSpec((1, 1, d), lambda i, idx_ref: (idx_ref[i], 0, 0)),
        ],
        out_specs=pl.BlockSpec((1, 1, d), lambda i, idx_ref: (i, 0, 0)),
    )
    out = pl.pallas_call(
        _expert_kernel,
        grid_spec=grid_spec,
        out_shape=jax.ShapeDtypeStruct((B, 1, d), jnp.float32),
    )(idx, rep.reshape(B, 1, d), We, be.reshape(E, 1, d))
    return out.reshape(B, d)


def _head(eo, Wh, bh, bn=4096):
    B, K = eo.shape
    V = Wh.shape[0]
    return pl.pallas_call(
        functools.partial(_mm_bias_kernel, relu=False),
        grid=(pl.cdiv(V, bn),),
        in_specs=[
            pl.BlockSpec((B, K), lambda j: (0, 0)),
            pl.BlockSpec((bn, K), lambda j: (j, 0)),
            pl.BlockSpec((1, bn), lambda j: (0, j)),
        ],
        out_specs=pl.BlockSpec((B, bn), lambda j: (0, j)),
        out_shape=jax.ShapeDtypeStruct((B, V), jnp.float32),
    )(eo, Wh, bh.reshape(1, V))


def kernel(x, Wqkv, bqkv, Wo, bo, ln1g, ln1b, W1, b1, W2, b2, ln2g, ln2b,
           Wg, bg, We, be, Wh, bh):
    S, B, d = x.shape
    qkv = _mm_bias_bmajor(x, Wqkv[0], bqkv[0])
    ao = _attn(qkv)
    h, qkv2 = _block(ao, Wo[0], bo[0], x, ln1g[0], ln1b[0], W1[0], b1[0],
                     W2[0], b2[0], ln2g[0], ln2b[0], Wqkv[1], bqkv[1],
                     res_is_sbview=True)
    ao2 = _attn(qkv2)
    rep, _ = _block(ao2, Wo[1], bo[1], h, ln1g[1], ln1b[1], W1[1], b1[1],
                    W2[1], b2[1], ln2g[1], ln2b[1])
    gw, idx = _gate(rep, Wg, bg)
    eo = _expert(idx, rep, We, be)
    logits = _head(eo, Wh, bh)
    return logits, gw, idx
